# speedup vs baseline: 1.0006x; 1.0006x over previous
import functools

import jax
import jax.numpy as jnp
from jax.experimental import pallas as pl
from jax.experimental.pallas import tpu as pltpu

N = 10000
D = 256
K = 3
NEG = 0.2


def _leaky(v):
    return jnp.where(v >= 0, v, NEG * v)


def _fc_body(x_ref, wt_ref, al_ref, ar_ref, feat_ref, el_ref, er_ref):
    x = x_ref[...]
    wt = wt_ref[...]
    f = jnp.dot(x, wt, preferred_element_type=jnp.float32)
    feat_ref[...] = f
    el_ref[...] = f @ al_ref[...]
    er_ref[...] = f @ ar_ref[...]


def _fc_stage(x, wt, al_col, ar_col):
    B = 2000
    return pl.pallas_call(
        _fc_body,
        grid=(N // B,),
        in_specs=[
            pl.BlockSpec((B, D), lambda i: (i, 0)),
            pl.BlockSpec((D, D), lambda i: (0, 0)),
            pl.BlockSpec((D, 128), lambda i: (0, 0)),
            pl.BlockSpec((D, 128), lambda i: (0, 0)),
        ],
        out_specs=[
            pl.BlockSpec((B, D), lambda i: (i, 0)),
            pl.BlockSpec((B, 128), lambda i: (i, 0)),
            pl.BlockSpec((B, 128), lambda i: (i, 0)),
        ],
        out_shape=[
            jax.ShapeDtypeStruct((N, D), jnp.float32),
            jax.ShapeDtypeStruct((N, 128), jnp.float32),
            jax.ShapeDtypeStruct((N, 128), jnp.float32),
        ],
    )(x, wt, al_col, ar_col)


def kernel(x, edge_index, W_fc, attn_l, attn_r, hop_attn_l, hop_attn_r, position_emb, bias):
    src = edge_index[0]
    dst = edge_index[1]
    al_col = jnp.broadcast_to(attn_l.reshape(D, 1), (D, 128))
    ar_col = jnp.broadcast_to(attn_r.reshape(D, 1), (D, 128))
    feat, el_w, er_w = _fc_stage(x, W_fc.T, al_col, ar_col)
    el = el_w[:, :1]
    er = er_w[:, :1]

    e = _leaky(el[src] + er[dst])
    m = jax.ops.segment_max(e, dst, num_segments=N)
    m = jnp.where(jnp.isfinite(m), m, 0.0)
    ee = jnp.exp(e - m[dst])
    den = jax.ops.segment_sum(ee, dst, num_segments=N)
    a = ee / den[dst]

    h = feat
    hstack = [feat]
    for _ in range(K):
        msg = h[src] * a
        h = jax.ops.segment_sum(msg, dst, num_segments=N)
        hstack.append(h)

    pe = position_emb.reshape(K + 1, D)
    hstack = [hstack[i] + pe[i][None, :] for i in range(K + 1)]
    hal = hop_attn_l.reshape(D)
    har = hop_attn_r.reshape(D)
    a_r = hstack[0] @ har
    logits = jnp.stack([hk @ hal + a_r for hk in hstack], axis=-1)
    logits = _leaky(logits)
    w = jax.nn.softmax(logits, axis=-1)
    rst = sum(hstack[i] * w[:, i:i + 1] for i in range(K + 1))
    rst = rst + bias.reshape(1, D)
    return rst.reshape(N, 1, D)


# speedup vs baseline: 5.3562x; 5.3529x over previous
import functools

import jax
import jax.numpy as jnp
from jax import lax
from jax.experimental import pallas as pl
from jax.experimental.pallas import tpu as pltpu
from jax.experimental.pallas import tpu_sc as plsc

N = 10000
E = 160000
D = 256
HD = 128
K = 3
NEG = 0.2

CH = 128
NCHUNK = E // CH
NCORE = 2
NSUB = 16
L = 16

_VMESH = plsc.VectorSubcoreMesh(core_axis_name="c", subcore_axis_name="s")

_TSLICES = ((0, 128), (128, 128), (256, 128), (384, 128), (512, 112))

import dataclasses as _dc
_SC_CP = pltpu.CompilerParams()
if "needs_layout_passes" in pltpu.CompilerParams.__dataclass_fields__:
    _SC_CP = _dc.replace(_SC_CP, needs_layout_passes=False)


def _leaky(v):
    return jnp.where(v >= 0, v, NEG * v)


def _fc_body(x_ref, wt_ref, al_ref, ar_ref, feat_ref, el_ref, er_ref):
    x = x_ref[...]
    f = jnp.dot(x, wt_ref[...], preferred_element_type=jnp.float32)
    feat_ref[...] = f
    el_ref[...] = f @ al_ref[...]
    er_ref[...] = f @ ar_ref[...]


def _fc_stage(x, wt, al_col, ar_col):
    B = 2000
    return pl.pallas_call(
        _fc_body,
        grid=(N // B,),
        in_specs=[
            pl.BlockSpec((B, D), lambda i: (i, 0)),
            pl.BlockSpec((D, D), lambda i: (0, 0)),
            pl.BlockSpec((D, 128), lambda i: (0, 0)),
            pl.BlockSpec((D, 128), lambda i: (0, 0)),
        ],
        out_specs=[
            pl.BlockSpec((B, D), lambda i: (i, 0)),
            pl.BlockSpec((B, 128), lambda i: (i, 0)),
            pl.BlockSpec((B, 128), lambda i: (i, 0)),
        ],
        out_shape=[
            jax.ShapeDtypeStruct((N, D), jnp.float32),
            jax.ShapeDtypeStruct((N, 128), jnp.float32),
            jax.ShapeDtypeStruct((N, 128), jnp.float32),
        ],
    )(x, wt, al_col, ar_col)


def _edge_body(el_h, er_h, src_h, dst_h, b_h, ee_h, den_h,
               el_v, er_v, b_v, src_v, dst_v, ee_v, zero_v, den_sh):
    c = lax.axis_index("c")
    s = lax.axis_index("s")
    w = c * NSUB + s

    pltpu.sync_copy(el_h, el_v)
    pltpu.sync_copy(er_h, er_v)
    pltpu.sync_copy(b_h, b_v)

    @pl.loop(0, 64)
    def _(i):
        zero_v[pl.ds(i * L, L)] = jnp.zeros((L,), jnp.float32)

    @pl.when(s < 10)
    def _():
        pltpu.sync_copy(zero_v.at[pl.ds(0, 1000)], den_sh.at[pl.ds(s * 1000, 1000)])

    plsc.subcore_barrier()

    bvec = b_v[...]

    @pl.loop(w, NCHUNK, step=NCORE * NSUB)
    def _(chunk):
        base = chunk * CH
        pltpu.sync_copy(src_h.at[pl.ds(base, CH)], src_v)
        pltpu.sync_copy(dst_h.at[pl.ds(base, CH)], dst_v)
        for j in range(CH // L):
            sl = pl.ds(j * L, L)
            s16 = src_v[sl]
            d16 = dst_v[sl]
            e = plsc.load_gather(el_v, [s16]) + plsc.load_gather(er_v, [d16])
            e = jnp.where(e >= 0, e, NEG * e)
            ee_v[sl] = jnp.exp(e - bvec)
        pltpu.sync_copy(ee_v, ee_h.at[pl.ds(base, CH)])
        pltpu.sync_copy(ee_v, den_sh.at[dst_v], add=True)

    plsc.subcore_barrier()

    @pl.when(s < 10)
    def _():
        pltpu.sync_copy(den_sh.at[pl.ds(s * 1000, 1000)], zero_v.at[pl.ds(0, 1000)])
        pltpu.sync_copy(zero_v.at[pl.ds(0, 1000)],
                        den_h.at[pl.ds(c * N + s * 1000, 1000)])


def _edge_stage(el, er, src, dst, b_arr):
    f = pl.kernel(
        _edge_body,
        out_type=[
            jax.ShapeDtypeStruct((E,), jnp.float32),
            jax.ShapeDtypeStruct((NCORE * N,), jnp.float32),
        ],
        mesh=_VMESH,
        compiler_params=_SC_CP,
        scratch_types=[
            pltpu.VMEM((N,), jnp.float32),
            pltpu.VMEM((N,), jnp.float32),
            pltpu.VMEM((L,), jnp.float32),
            pltpu.VMEM((CH,), jnp.int32),
            pltpu.VMEM((CH,), jnp.int32),
            pltpu.VMEM((CH,), jnp.float32),
            pltpu.VMEM((1024,), jnp.float32),
            pltpu.VMEM_SHARED((N,), jnp.float32),
        ],
    )
    return f(el, er, src, dst, b_arr)


def _norm_body(den_h, dst_h, ee_h, a_h, d0_v, d1_v, dst_v, ee_v, a_v):
    c = lax.axis_index("c")
    s = lax.axis_index("s")
    w = c * NSUB + s

    pltpu.sync_copy(den_h.at[pl.ds(0, N)], d0_v)
    pltpu.sync_copy(den_h.at[pl.ds(N, N)], d1_v)

    @pl.loop(0, N // L)
    def _(i):
        sl = pl.ds(i * L, L)
        d0_v[sl] = d0_v[sl] + d1_v[sl]

    @pl.loop(w, NCHUNK, step=NCORE * NSUB)
    def _(chunk):
        base = chunk * CH
        pltpu.sync_copy(dst_h.at[pl.ds(base, CH)], dst_v)
        pltpu.sync_copy(ee_h.at[pl.ds(base, CH)], ee_v)
        for j in range(CH // L):
            sl = pl.ds(j * L, L)
            d16 = dst_v[sl]
            a_v[sl] = ee_v[sl] / plsc.load_gather(d0_v, [d16])
        pltpu.sync_copy(a_v, a_h.at[pl.ds(base, CH)])


def _norm_stage(den2, dst, ee):
    f = pl.kernel(
        _norm_body,
        out_type=jax.ShapeDtypeStruct((E,), jnp.float32),
        mesh=_VMESH,
        compiler_params=_SC_CP,
        scratch_types=[
            pltpu.VMEM((N,), jnp.float32),
            pltpu.VMEM((N,), jnp.float32),
            pltpu.VMEM((CH,), jnp.int32),
            pltpu.VMEM((CH,), jnp.float32),
            pltpu.VMEM((CH,), jnp.float32),
        ],
    )
    return f(den2, dst, ee)


def _hop_body(h2_h, src_h, dst_h, a_h, hn2_h,
              src_v, dst_v, a_v, gidx_v, g_v, acc_sh):
    c = lax.axis_index("c")
    s = lax.axis_index("s")
    coff = c * N

    @pl.loop(0, CH)
    def _(i):
        row = g_v.at[i]
        for j in range(HD // L):
            row[pl.ds(j * L, L)] = jnp.zeros((L,), jnp.float32)

    for off, sz in _TSLICES:
        pltpu.sync_copy(g_v.at[pl.ds(0, sz)],
                        acc_sh.at[pl.ds(s * 624 + off, sz)])

    @pl.when(s == NSUB - 1)
    def _():
        pltpu.sync_copy(g_v.at[pl.ds(0, 16)], acc_sh.at[pl.ds(9984, 16)])

    plsc.subcore_barrier()

    @pl.loop(s, NCHUNK, step=NSUB)
    def _(chunk):
        base = chunk * CH
        pltpu.sync_copy(src_h.at[pl.ds(base, CH)], src_v)
        pltpu.sync_copy(dst_h.at[pl.ds(base, CH)], dst_v)
        pltpu.sync_copy(a_h.at[pl.ds(base, CH)], a_v)
        for j in range(CH // L):
            sl = pl.ds(j * L, L)
            gidx_v[sl] = src_v[sl] + coff

        pltpu.sync_copy(h2_h.at[gidx_v], g_v)

        @pl.loop(0, CH)
        def _(i):
            bc = plsc.load_gather(a_v, [jnp.full((L,), i, jnp.int32)])
            row = g_v.at[i]
            for j in range(HD // L):
                sl2 = pl.ds(j * L, L)
                row[sl2] = row[sl2] * bc

        pltpu.sync_copy(g_v, acc_sh.at[dst_v], add=True)

    plsc.subcore_barrier()

    for off, sz in _TSLICES:
        row0 = s * 624 + off
        pltpu.sync_copy(acc_sh.at[pl.ds(row0, sz)], g_v.at[pl.ds(0, sz)])
        pltpu.sync_copy(g_v.at[pl.ds(0, sz)], hn2_h.at[pl.ds(coff + row0, sz)])

    @pl.when(s == NSUB - 1)
    def _():
        pltpu.sync_copy(acc_sh.at[pl.ds(9984, 16)], g_v.at[pl.ds(0, 16)])
        pltpu.sync_copy(g_v.at[pl.ds(0, 16)], hn2_h.at[pl.ds(coff + 9984, 16)])


def _hop_stage(h2, src, dst, a):
    f = pl.kernel(
        _hop_body,
        out_type=jax.ShapeDtypeStruct((NCORE * N, HD), jnp.float32),
        mesh=_VMESH,
        compiler_params=_SC_CP,
        scratch_types=[
            pltpu.VMEM((CH,), jnp.int32),
            pltpu.VMEM((CH,), jnp.int32),
            pltpu.VMEM((CH,), jnp.float32),
            pltpu.VMEM((CH,), jnp.int32),
            pltpu.VMEM((CH, HD), jnp.float32),
            pltpu.VMEM_SHARED((N, HD), jnp.float32),
        ],
    )
    return f(h2, src, dst, a)


def _combine_body(h0a, h0b, h1a, h1b, h2a, h2b, h3a, h3b, p_ref, out_ref):
    p = p_ref[...]
    ha = [h0a[...], h1a[...], h2a[...], h3a[...]]
    hb = [h0b[...], h1b[...], h2b[...], h3b[...]]
    for k in range(K + 1):
        ha[k] = ha[k] + p[k:k + 1, :HD]
        hb[k] = hb[k] + p[k:k + 1, HD:]
    hal_a, hal_b = p[4:5, :HD], p[4:5, HD:]
    har_a, har_b = p[5:6, :HD], p[5:6, HD:]
    a_r = jnp.sum(ha[0] * har_a, axis=1, keepdims=True) + \
        jnp.sum(hb[0] * har_b, axis=1, keepdims=True)
    logits = [jnp.sum(ha[k] * hal_a, axis=1, keepdims=True) +
              jnp.sum(hb[k] * hal_b, axis=1, keepdims=True) + a_r
              for k in range(K + 1)]
    logits = [_leaky(lg) for lg in logits]
    mx = functools.reduce(jnp.maximum, logits)
    exps = [jnp.exp(lg - mx) for lg in logits]
    den = functools.reduce(jnp.add, exps)
    outa = functools.reduce(jnp.add, [ha[k] * (exps[k] / den) for k in range(K + 1)])
    outb = functools.reduce(jnp.add, [hb[k] * (exps[k] / den) for k in range(K + 1)])
    out_ref[:, :HD] = outa + p[6:7, :HD]
    out_ref[:, HD:] = outb + p[6:7, HD:]


def _combine_stage(hs2, params):
    B = 2000
    NB = N // B
    ins = []
    specs = []
    for h2 in hs2:
        ins.append(h2)
        specs.append(pl.BlockSpec((B, HD), lambda i: (i, 0)))
        ins.append(h2)
        specs.append(pl.BlockSpec((B, HD), lambda i: (NB + i, 0)))
    ins.append(params)
    specs.append(pl.BlockSpec((8, D), lambda i: (0, 0)))
    return pl.pallas_call(
        _combine_body,
        grid=(NB,),
        in_specs=specs,
        out_specs=pl.BlockSpec((B, D), lambda i: (i, 0)),
        out_shape=jax.ShapeDtypeStruct((N, D), jnp.float32),
    )(*ins)


def kernel(x, edge_index, W_fc, attn_l, attn_r, hop_attn_l, hop_attn_r, position_emb, bias):
    src = edge_index[0]
    dst = edge_index[1]
    al_col = jnp.broadcast_to(attn_l.reshape(D, 1), (D, 128))
    ar_col = jnp.broadcast_to(attn_r.reshape(D, 1), (D, 128))
    feat, el_w, er_w = _fc_stage(x, W_fc.T, al_col, ar_col)
    el = el_w[:, 0]
    er = er_w[:, 0]

    b_const = jnp.max(el) + jnp.max(er)
    b_arr = jnp.full((L,), b_const, jnp.float32)

    ee, den2 = _edge_stage(el, er, src, dst, b_arr)
    a = _norm_stage(den2, dst, ee)

    h2 = jnp.concatenate([feat[:, :HD], feat[:, HD:]], axis=0)
    hs2 = [h2]
    for _ in range(K):
        h2 = _hop_stage(h2, src, dst, a)
        hs2.append(h2)

    pe = position_emb.reshape(K + 1, D)
    params = jnp.concatenate([
        pe,
        hop_attn_l.reshape(1, D),
        hop_attn_r.reshape(1, D),
        bias.reshape(1, D),
        jnp.zeros((1, D), jnp.float32),
    ], axis=0)
    rst = _combine_stage(hs2, params)
    return rst.reshape(N, 1, D)
